# baseline (speedup 1.0000x reference)
import jax
import jax.numpy as jnp
from jax.experimental import pallas as pl
from jax.experimental.pallas import tpu as pltpu

_ROWS = 256


def _bcast_body(tab_ref, out_ref):
    out_ref[0] = tab_ref[...]


def kernel(x, table):
    batch = x.shape[0]
    n_rows, embed = table.shape
    return pl.pallas_call(
        _bcast_body,
        grid=(n_rows // _ROWS, batch),
        in_specs=[pl.BlockSpec((_ROWS, embed), lambda r, b: (r, 0))],
        out_specs=pl.BlockSpec((1, _ROWS, embed), lambda r, b: (b, r, 0)),
        out_shape=jax.ShapeDtypeStruct((batch, n_rows, embed), table.dtype),
        compiler_params=pltpu.CompilerParams(
            dimension_semantics=("parallel", "parallel"),
        ),
    )(table)
